# baseline (speedup 1.0000x reference)
import functools

import jax
import jax.numpy as jnp
import numpy as np
from jax import lax
from jax.experimental import pallas as pl
from jax.experimental.pallas import tpu as pltpu
from jax.experimental.pallas import tpu_sc as plsc

_HEADS = [8, 8, 8, 1]
_OUTC = [128, 64, 32, 1]
_DIMS = [128, 1024, 512, 256]
_N = 10000
_NP = 10240
_E = 320000
_NW = 32
_EPT = _E // _NW
_EB = 80
_NBLK = _EPT // _EB
_EG = 2000
_BPG = _EG // _EB
_NG = _EPT // _EG
_LROWS = _NP // 16

_mesh = plsc.VectorSubcoreMesh(core_axis_name="c", subcore_axis_name="s")


def _wid():
    return lax.axis_index("c") * 16 + lax.axis_index("s")


def _mm_body(x_ref, w_ref, asrc_ref, adst_ref, h_ref, as_ref, ad_ref):
    h = jnp.dot(x_ref[...], w_ref[...], preferred_element_type=jnp.float32)
    h_ref[...] = h
    as_ref[...] = jnp.dot(h, asrc_ref[...], preferred_element_type=jnp.float32)
    ad_ref[...] = jnp.dot(h, adst_ref[...], preferred_element_type=jnp.float32)


def _tc_matmul(xin, w, asrc_m, adst_m):
    din = xin.shape[1]
    hc = w.shape[1]
    bn = 256
    grid = (_NP // bn,)
    return pl.pallas_call(
        _mm_body,
        grid=grid,
        in_specs=[
            pl.BlockSpec((bn, din), lambda i: (i, 0)),
            pl.BlockSpec((din, hc), lambda i: (0, 0)),
            pl.BlockSpec((hc, 128), lambda i: (0, 0)),
            pl.BlockSpec((hc, 128), lambda i: (0, 0)),
        ],
        out_specs=[
            pl.BlockSpec((bn, hc), lambda i: (i, 0)),
            pl.BlockSpec((bn, 128), lambda i: (i, 0)),
            pl.BlockSpec((bn, 128), lambda i: (i, 0)),
        ],
        out_shape=[
            jax.ShapeDtypeStruct((_NP, hc), jnp.float32),
            jax.ShapeDtypeStruct((_NP, 128), jnp.float32),
            jax.ShapeDtypeStruct((_NP, 128), jnp.float32),
        ],
    )(xin, w, asrc_m, adst_m)


def _cvec_body(we_ref, ae_ref, o_ref):
    o_ref[...] = jnp.dot(we_ref[...], ae_ref[...],
                         preferred_element_type=jnp.float32)


def _tc_cvec(we_pad, ae_m):
    hc = we_pad.shape[1]
    return pl.pallas_call(
        _cvec_body,
        out_shape=jax.ShapeDtypeStruct((8, 128), jnp.float32),
    )(we_pad, ae_m)


def _sum_parts_body(p_ref, o_ref):
    o_ref[...] = jnp.sum(p_ref[...], axis=0)


def _tc_sum_parts(parts):
    k, h, n = parts.shape
    bn = 1280
    return pl.pallas_call(
        _sum_parts_body,
        grid=(n // bn,),
        in_specs=[pl.BlockSpec((k, h, bn), lambda i: (0, 0, i))],
        out_specs=pl.BlockSpec((h, bn), lambda i: (0, i)),
        out_shape=jax.ShapeDtypeStruct((h, n), jnp.float32),
    )(parts)


def _combine_body(apply_relu, p0_ref, p1_ref, b_ref, o_ref):
    v = p0_ref[...] + p1_ref[...] + b_ref[...]
    if apply_relu:
        v = jnp.maximum(v, 0.0)
    o_ref[...] = v


def _tc_combine(p0, p1, b2d, apply_relu):
    f = p0.shape[1]
    bn = 256
    return pl.pallas_call(
        functools.partial(_combine_body, apply_relu),
        grid=(_NP // bn,),
        in_specs=[
            pl.BlockSpec((bn, f), lambda i: (i, 0)),
            pl.BlockSpec((bn, f), lambda i: (i, 0)),
            pl.BlockSpec((1, f), lambda i: (0, 0)),
        ],
        out_specs=pl.BlockSpec((bn, f), lambda i: (i, 0)),
        out_shape=jax.ShapeDtypeStruct((_NP, f), jnp.float32),
    )(p0, p1, b2d)


def _sum32_body(p_ref, b_ref, o_ref):
    o_ref[...] = jnp.sum(p_ref[...], axis=0) + b_ref[...]


def _tc_sum32_bias(parts, b2d):
    k = parts.shape[0]
    r = parts.shape[1]
    br = 16
    return pl.pallas_call(
        _sum32_body,
        grid=(r // br,),
        in_specs=[
            pl.BlockSpec((k, br, 128), lambda i: (0, i, 0)),
            pl.BlockSpec((1, 128), lambda i: (0, 0)),
        ],
        out_specs=pl.BlockSpec((br, 128), lambda i: (i, 0)),
        out_shape=jax.ShapeDtypeStruct((r, 128), jnp.float32),
    )(parts, b2d)


def _zero_ref(ref, n):
    def body(i, _):
        ref[pl.ds(i * 16, 16)] = jnp.zeros((16,), jnp.float32)
        return 0
    lax.fori_loop(0, n // 16, body, 0)


def _make_logits_kernel(H):
    @functools.partial(
        pl.kernel,
        out_type=(
            jax.ShapeDtypeStruct((H, _E), jnp.float32),
            jax.ShapeDtypeStruct((_NW, H, _NP), jnp.float32),
        ),
        mesh=_mesh,
        compiler_params=pltpu.CompilerParams(use_tc_tiling_on_sc=False, needs_layout_passes=False),
        scratch_types=[
            pltpu.VMEM((_EPT,), jnp.int32),
            pltpu.VMEM((_EPT,), jnp.int32),
            pltpu.VMEM((_EPT,), jnp.float32),
            pltpu.VMEM((_EPT,), jnp.float32),
            pltpu.VMEM((_NP,), jnp.float32),
            pltpu.VMEM((_NP,), jnp.float32),
            pltpu.VMEM((_NP,), jnp.float32),
            pltpu.VMEM((16,), jnp.float32),
        ],
    )
    def k(src_hbm, dst_hbm, asrcT_hbm, adstT_hbm, ea_hbm, c_hbm,
          sT_hbm, dpart_hbm,
          src_v, dst_v, ea_v, s_v, asrc_v, adst_v, den_v, c_v):
        wid = _wid()
        ebase = wid * _EPT
        pltpu.sync_copy(src_hbm.at[pl.ds(ebase, _EPT)], src_v)
        pltpu.sync_copy(dst_hbm.at[pl.ds(ebase, _EPT)], dst_v)
        pltpu.sync_copy(ea_hbm.at[pl.ds(ebase, _EPT)], ea_v)
        pltpu.sync_copy(c_hbm, c_v)
        for h in range(H):
            pltpu.sync_copy(asrcT_hbm.at[h], asrc_v)
            pltpu.sync_copy(adstT_hbm.at[h], adst_v)
            _zero_ref(den_v, _NP)
            ch = plsc.load_gather(c_v, [jnp.full((16,), h, jnp.int32)])

            def body(i, _):
                sl = pl.ds(i * 16, 16)
                idst = dst_v[sl]
                va = plsc.load_gather(asrc_v, [src_v[sl]])
                vb = plsc.load_gather(adst_v, [idst])
                al = va + vb + ea_v[sl] * ch
                al = jnp.where(al > 0.0, al, al * jnp.float32(0.2))
                sv = jnp.exp(al)
                s_v[sl] = sv
                plsc.addupdate_scatter(den_v, [idst], sv)
                return 0

            lax.fori_loop(0, _EPT // 16, body, 0, unroll=8)
            pltpu.sync_copy(s_v, sT_hbm.at[h, pl.ds(ebase, _EPT)])
            pltpu.sync_copy(den_v, dpart_hbm.at[wid, h])

    return k


def _make_weights_kernel(H):
    @functools.partial(
        pl.kernel,
        out_type=jax.ShapeDtypeStruct((H, _E), jnp.float32),
        mesh=_mesh,
        compiler_params=pltpu.CompilerParams(use_tc_tiling_on_sc=False, needs_layout_passes=False),
        scratch_types=[
            pltpu.VMEM((_EPT,), jnp.int32),
            pltpu.VMEM((_EPT,), jnp.float32),
            pltpu.VMEM((_EPT,), jnp.float32),
            pltpu.VMEM((_NP,), jnp.float32),
        ],
    )
    def k(dst_hbm, sT_hbm, den_hbm, wT_hbm, dst_v, s_v, w_v, den_v):
        wid = _wid()
        ebase = wid * _EPT
        pltpu.sync_copy(dst_hbm.at[pl.ds(ebase, _EPT)], dst_v)
        for h in range(H):
            pltpu.sync_copy(sT_hbm.at[h, pl.ds(ebase, _EPT)], s_v)
            pltpu.sync_copy(den_hbm.at[h], den_v)

            def body(i, _):
                sl = pl.ds(i * 16, 16)
                d = plsc.load_gather(den_v, [dst_v[sl]])
                w_v[sl] = s_v[sl] / (d + jnp.float32(1e-16))
                return 0

            lax.fori_loop(0, _EPT // 16, body, 0, unroll=8)
            pltpu.sync_copy(w_v, wT_hbm.at[h, pl.ds(ebase, _EPT)])

    return k


def _make_message_kernel(C, H128):
    @functools.partial(
        pl.kernel,
        out_type=jax.ShapeDtypeStruct((2, _NP, 128), jnp.float32),
        mesh=_mesh,
        compiler_params=pltpu.CompilerParams(use_tc_tiling_on_sc=False, needs_layout_passes=False),
        scratch_types=[
            pltpu.VMEM((_BPG, _EB), jnp.int32),
            pltpu.VMEM((_BPG, _EB), jnp.int32),
            pltpu.VMEM((H128, _EG), jnp.float32),
            pltpu.VMEM((_EB, 128), jnp.float32),
            pltpu.VMEM((_EB, 128), jnp.float32),
            pltpu.VMEM((_EB, 128), jnp.float32),
            pltpu.VMEM_SHARED((_NP, 128), jnp.float32),
            pltpu.SemaphoreType.DMA,
            pltpu.SemaphoreType.DMA,
        ],
    )
    def k(h_hbm, wT_hbm, src3_hbm, dst3_hbm, opart_hbm,
          src_g, dst_g, w_v, rows_a, rows_b, rows_o, acc_sh, sem_a, sem_b):
        cid = lax.axis_index("c")
        sid = lax.axis_index("s")
        wid = cid * 16 + sid
        ebase = wid * _EPT

        _zero2d(rows_a, _EB)
        r0 = sid * _LROWS

        def zbody(j, _):
            pltpu.sync_copy(rows_a, acc_sh.at[pl.ds(r0 + j * _EB, _EB)])
            return 0

        lax.fori_loop(0, _LROWS // _EB, zbody, 0)
        plsc.subcore_barrier()

        def scale_and_scatter(jb, rows_v):
            @plsc.parallel_loop(0, _EB, unroll=4)
            def edge(l):
                e = jb * _EB + l
                idxe = jnp.full((16,), 0, jnp.int32) + e
                for hh in range(H128):
                    ws = plsc.load_gather(
                        w_v, [jnp.full((16,), hh, jnp.int32), idxe])
                    for p in range(C // 16):
                        csl = pl.ds(hh * C + p * 16, 16)
                        rows_o[l, csl] = rows_v[l, csl] * ws

            pltpu.sync_copy(rows_o, acc_sh.at[dst_g.at[jb]], add=True)

        def group(g, _):
            pltpu.sync_copy(src3_hbm.at[wid, pl.ds(g * _BPG, _BPG)], src_g)
            pltpu.sync_copy(dst3_hbm.at[wid, pl.ds(g * _BPG, _BPG)], dst_g)
            for hh in range(H128):
                pltpu.sync_copy(wT_hbm.at[hh, pl.ds(ebase + g * _EG, _EG)],
                                w_v.at[hh])

            pltpu.async_copy(h_hbm.at[src_g.at[0]], rows_a, sem_a)

            def pair(p, _):
                j0 = 2 * p
                pltpu.async_copy(h_hbm.at[src_g.at[j0 + 1]], rows_b, sem_b)
                pltpu.make_async_copy(
                    h_hbm.at[src_g.at[j0]], rows_a, sem_a).wait()
                scale_and_scatter(j0, rows_a)
                pltpu.async_copy(h_hbm.at[src_g.at[j0 + 2]], rows_a, sem_a)
                pltpu.make_async_copy(
                    h_hbm.at[src_g.at[j0 + 1]], rows_b, sem_b).wait()
                scale_and_scatter(j0 + 1, rows_b)
                return 0

            lax.fori_loop(0, (_BPG - 1) // 2, pair, 0)
            pltpu.make_async_copy(
                h_hbm.at[src_g.at[_BPG - 1]], rows_a, sem_a).wait()
            scale_and_scatter(_BPG - 1, rows_a)
            return 0

        lax.fori_loop(0, _NG, group, 0)
        plsc.subcore_barrier()
        pltpu.sync_copy(acc_sh.at[pl.ds(r0, _LROWS)],
                        opart_hbm.at[cid, pl.ds(r0, _LROWS)])

    return k


def _zero2d(ref, nrows):
    def body(i, _):
        for p in range(8):
            ref[i, pl.ds(p * 16, 16)] = jnp.zeros((16,), jnp.float32)
        return 0
    lax.fori_loop(0, nrows, body, 0)


def _make_scalar_message_kernel():
    @functools.partial(
        pl.kernel,
        out_type=jax.ShapeDtypeStruct((_NW, _NP), jnp.float32),
        mesh=_mesh,
        compiler_params=pltpu.CompilerParams(use_tc_tiling_on_sc=False, needs_layout_passes=False),
        scratch_types=[
            pltpu.VMEM((_EPT,), jnp.int32),
            pltpu.VMEM((_EPT,), jnp.int32),
            pltpu.VMEM((_EPT,), jnp.float32),
            pltpu.VMEM((_NP,), jnp.float32),
            pltpu.VMEM((_NP,), jnp.float32),
        ],
    )
    def k(hcol_hbm, w_hbm, src_hbm, dst_hbm, opart_hbm,
          src_v, dst_v, w_v, hc_v, acc_v):
        wid = _wid()
        ebase = wid * _EPT
        pltpu.sync_copy(src_hbm.at[pl.ds(ebase, _EPT)], src_v)
        pltpu.sync_copy(dst_hbm.at[pl.ds(ebase, _EPT)], dst_v)
        pltpu.sync_copy(w_hbm.at[pl.ds(ebase, _EPT)], w_v)
        pltpu.sync_copy(hcol_hbm, hc_v)
        _zero_ref(acc_v, _NP)

        def body(i, _):
            sl = pl.ds(i * 16, 16)
            v = plsc.load_gather(hc_v, [src_v[sl]]) * w_v[sl]
            plsc.addupdate_scatter(acc_v, [dst_v[sl]], v)
            return 0

        lax.fori_loop(0, _EPT // 16, body, 0, unroll=8)
        pltpu.sync_copy(acc_v, opart_hbm.at[wid])

    return k


_logits8 = _make_logits_kernel(8)
_logits1 = _make_logits_kernel(1)
_weights8 = _make_weights_kernel(8)
_weights1 = _make_weights_kernel(1)
_msg_by_c = {128: _make_message_kernel(128, 1),
             64: _make_message_kernel(64, 2),
             32: _make_message_kernel(32, 4)}
_msg_scalar = _make_scalar_message_kernel()


def _expand_att(att):
    h, c = att.shape
    rows = np.arange(h * c)
    cols = np.repeat(np.arange(h), c)
    m = jnp.zeros((h * c, 128), jnp.float32)
    return m.at[rows, cols].set(att.reshape(-1))


def kernel(x, edge_index, edge_attr, params):
    src = edge_index[0].astype(jnp.int32)
    dst = edge_index[1].astype(jnp.int32)
    ea = edge_attr[:, 0].astype(jnp.float32)
    src3 = src.reshape(_NW, _NBLK, _EB)
    dst3 = dst.reshape(_NW, _NBLK, _EB)

    xin = jnp.pad(x, ((0, _NP - _N), (0, 0)))
    for i in range(4):
        H, C = _HEADS[i], _OUTC[i]
        hc = H * C
        w_mat = params['W%d' % i]
        if hc < 128:
            w_mat = jnp.pad(w_mat, ((0, 0), (0, 128 - hc)))
        asrc_m = _expand_att(params['att_src%d' % i])
        adst_m = _expand_att(params['att_dst%d' % i])
        if hc < 128:
            asrc_m = jnp.pad(asrc_m, ((0, 128 - hc), (0, 0)))
            adst_m = jnp.pad(adst_m, ((0, 128 - hc), (0, 0)))
        h_full, a_src, a_dst = _tc_matmul(xin, w_mat, asrc_m, adst_m)

        ae_m = _expand_att(params['att_e%d' % i])
        we_pad = jnp.pad(params['We%d' % i], ((0, 7), (0, 0)))
        if hc < 128:
            ae_m = jnp.pad(ae_m, ((0, 128 - hc), (0, 0)))
            we_pad = jnp.pad(we_pad, ((0, 0), (0, 128 - hc)))
        c16 = _tc_cvec(we_pad, ae_m)[0, :16]

        asrcT = a_src[:, :H].T + 0.0
        adstT = a_dst[:, :H].T + 0.0

        logits_k = _logits8 if H == 8 else _logits1
        weights_k = _weights8 if H == 8 else _weights1
        sT, dparts = logits_k(src, dst, asrcT, adstT, ea, c16)
        denom = _tc_sum_parts(dparts)
        wT = weights_k(dst, sT, denom)

        if i < 3:
            msg_k = _msg_by_c[C]
            h128 = 128 // C
            p0_chunks, p1_chunks = [], []
            for j in range(hc // 128):
                h_chunk = h_full[:, j * 128:(j + 1) * 128] + 0.0
                w_chunk = wT[j * h128:(j + 1) * h128] + 0.0
                opart = msg_k(h_chunk, w_chunk, src3, dst3)
                p0_chunks.append(opart[0])
                p1_chunks.append(opart[1])
            p0 = jnp.concatenate(p0_chunks, axis=1)
            p1 = jnp.concatenate(p1_chunks, axis=1)
            b2d = params['b%d' % i][None, :]
            xin = _tc_combine(p0, p1, b2d, True)
        else:
            hcol = h_full[:, 0] + 0.0
            parts = _msg_scalar(hcol, wT[0], src, dst)
            b2d = jnp.full((1, 128), params['b%d' % i][0], jnp.float32)
            out2d = _tc_sum32_bias(parts.reshape(_NW, _NP // 128, 128), b2d)
            return out2d.reshape(_NP)[:_N, None]


# speedup vs baseline: 1.0604x; 1.0604x over previous
import functools

import jax
import jax.numpy as jnp
import numpy as np
from jax import lax
from jax.experimental import pallas as pl
from jax.experimental.pallas import tpu as pltpu
from jax.experimental.pallas import tpu_sc as plsc

_HEADS = [8, 8, 8, 1]
_OUTC = [128, 64, 32, 1]
_DIMS = [128, 1024, 512, 256]
_N = 10000
_NP = 10240
_E = 320000
_NW = 32
_EPT = _E // _NW
_EB = 80
_NBLK = _EPT // _EB
_EG = 2000
_BPG = _EG // _EB
_NG = _EPT // _EG
_LROWS = _NP // 16

_mesh = plsc.VectorSubcoreMesh(core_axis_name="c", subcore_axis_name="s")


def _wid():
    return lax.axis_index("c") * 16 + lax.axis_index("s")


def _mm_body(x_ref, w_ref, asrc_ref, adst_ref, h_ref, as_ref, ad_ref):
    h = jnp.dot(x_ref[...], w_ref[...], preferred_element_type=jnp.float32)
    h_ref[...] = h
    as_ref[...] = jnp.dot(h, asrc_ref[...], preferred_element_type=jnp.float32)
    ad_ref[...] = jnp.dot(h, adst_ref[...], preferred_element_type=jnp.float32)


def _tc_matmul(xin, w, asrc_m, adst_m):
    din = xin.shape[1]
    hc = w.shape[1]
    bn = 256
    grid = (_NP // bn,)
    return pl.pallas_call(
        _mm_body,
        grid=grid,
        in_specs=[
            pl.BlockSpec((bn, din), lambda i: (i, 0)),
            pl.BlockSpec((din, hc), lambda i: (0, 0)),
            pl.BlockSpec((hc, 128), lambda i: (0, 0)),
            pl.BlockSpec((hc, 128), lambda i: (0, 0)),
        ],
        out_specs=[
            pl.BlockSpec((bn, hc), lambda i: (i, 0)),
            pl.BlockSpec((bn, 128), lambda i: (i, 0)),
            pl.BlockSpec((bn, 128), lambda i: (i, 0)),
        ],
        out_shape=[
            jax.ShapeDtypeStruct((_NP, hc), jnp.float32),
            jax.ShapeDtypeStruct((_NP, 128), jnp.float32),
            jax.ShapeDtypeStruct((_NP, 128), jnp.float32),
        ],
    )(xin, w, asrc_m, adst_m)


def _cvec_body(we_ref, ae_ref, o_ref):
    o_ref[...] = jnp.dot(we_ref[...], ae_ref[...],
                         preferred_element_type=jnp.float32)


def _tc_cvec(we_pad, ae_m):
    hc = we_pad.shape[1]
    return pl.pallas_call(
        _cvec_body,
        out_shape=jax.ShapeDtypeStruct((8, 128), jnp.float32),
    )(we_pad, ae_m)


def _sum_parts_body(p_ref, o_ref):
    o_ref[...] = jnp.sum(p_ref[...], axis=0)


def _tc_sum_parts(parts):
    k, h, n = parts.shape
    bn = 1280
    return pl.pallas_call(
        _sum_parts_body,
        grid=(n // bn,),
        in_specs=[pl.BlockSpec((k, h, bn), lambda i: (0, 0, i))],
        out_specs=pl.BlockSpec((h, bn), lambda i: (0, i)),
        out_shape=jax.ShapeDtypeStruct((h, n), jnp.float32),
    )(parts)


def _combine_body(apply_relu, p0_ref, p1_ref, b_ref, o_ref):
    v = p0_ref[...] + p1_ref[...] + b_ref[...]
    if apply_relu:
        v = jnp.maximum(v, 0.0)
    o_ref[...] = v


def _tc_combine(p0, p1, b2d, apply_relu):
    f = p0.shape[1]
    bn = 256
    return pl.pallas_call(
        functools.partial(_combine_body, apply_relu),
        grid=(_NP // bn,),
        in_specs=[
            pl.BlockSpec((bn, f), lambda i: (i, 0)),
            pl.BlockSpec((bn, f), lambda i: (i, 0)),
            pl.BlockSpec((1, f), lambda i: (0, 0)),
        ],
        out_specs=pl.BlockSpec((bn, f), lambda i: (i, 0)),
        out_shape=jax.ShapeDtypeStruct((_NP, f), jnp.float32),
    )(p0, p1, b2d)


def _sum32_body(p_ref, b_ref, o_ref):
    o_ref[...] = jnp.sum(p_ref[...], axis=0) + b_ref[...]


def _tc_sum32_bias(parts, b2d):
    k = parts.shape[0]
    r = parts.shape[1]
    br = 16
    return pl.pallas_call(
        _sum32_body,
        grid=(r // br,),
        in_specs=[
            pl.BlockSpec((k, br, 128), lambda i: (0, i, 0)),
            pl.BlockSpec((1, 128), lambda i: (0, 0)),
        ],
        out_specs=pl.BlockSpec((br, 128), lambda i: (i, 0)),
        out_shape=jax.ShapeDtypeStruct((r, 128), jnp.float32),
    )(parts, b2d)


def _zero_ref(ref, n):
    def body(i, _):
        ref[pl.ds(i * 16, 16)] = jnp.zeros((16,), jnp.float32)
        return 0
    lax.fori_loop(0, n // 16, body, 0)


def _make_logits_kernel(H):
    @functools.partial(
        pl.kernel,
        out_type=(
            jax.ShapeDtypeStruct((H, _E), jnp.float32),
            jax.ShapeDtypeStruct((_NW, H, _NP), jnp.float32),
        ),
        mesh=_mesh,
        compiler_params=pltpu.CompilerParams(use_tc_tiling_on_sc=False, needs_layout_passes=False),
        scratch_types=[
            pltpu.VMEM((_EPT,), jnp.int32),
            pltpu.VMEM((_EPT,), jnp.int32),
            pltpu.VMEM((_EPT,), jnp.float32),
            pltpu.VMEM((_EPT,), jnp.float32),
            pltpu.VMEM((_NP,), jnp.float32),
            pltpu.VMEM((_NP,), jnp.float32),
            pltpu.VMEM((_NP,), jnp.float32),
            pltpu.VMEM((16,), jnp.float32),
        ],
    )
    def k(src_hbm, dst_hbm, asrcT_hbm, adstT_hbm, ea_hbm, c_hbm,
          sT_hbm, dpart_hbm,
          src_v, dst_v, ea_v, s_v, asrc_v, adst_v, den_v, c_v):
        wid = _wid()
        ebase = wid * _EPT
        pltpu.sync_copy(src_hbm.at[pl.ds(ebase, _EPT)], src_v)
        pltpu.sync_copy(dst_hbm.at[pl.ds(ebase, _EPT)], dst_v)
        pltpu.sync_copy(ea_hbm.at[pl.ds(ebase, _EPT)], ea_v)
        pltpu.sync_copy(c_hbm, c_v)
        for h in range(H):
            pltpu.sync_copy(asrcT_hbm.at[h], asrc_v)
            pltpu.sync_copy(adstT_hbm.at[h], adst_v)
            _zero_ref(den_v, _NP)
            ch = plsc.load_gather(c_v, [jnp.full((16,), h, jnp.int32)])

            def body(i, _):
                sl = pl.ds(i * 16, 16)
                idst = dst_v[sl]
                va = plsc.load_gather(asrc_v, [src_v[sl]])
                vb = plsc.load_gather(adst_v, [idst])
                al = va + vb + ea_v[sl] * ch
                al = jnp.where(al > 0.0, al, al * jnp.float32(0.2))
                sv = jnp.exp(al)
                s_v[sl] = sv
                plsc.addupdate_scatter(den_v, [idst], sv)
                return 0

            lax.fori_loop(0, _EPT // 16, body, 0, unroll=8)
            pltpu.sync_copy(s_v, sT_hbm.at[h, pl.ds(ebase, _EPT)])
            pltpu.sync_copy(den_v, dpart_hbm.at[wid, h])

    return k


def _make_weights_kernel(H):
    @functools.partial(
        pl.kernel,
        out_type=jax.ShapeDtypeStruct((H, _E), jnp.float32),
        mesh=_mesh,
        compiler_params=pltpu.CompilerParams(use_tc_tiling_on_sc=False, needs_layout_passes=False),
        scratch_types=[
            pltpu.VMEM((_EPT,), jnp.int32),
            pltpu.VMEM((_EPT,), jnp.float32),
            pltpu.VMEM((_EPT,), jnp.float32),
            pltpu.VMEM((_NP,), jnp.float32),
        ],
    )
    def k(dst_hbm, sT_hbm, den_hbm, wT_hbm, dst_v, s_v, w_v, den_v):
        wid = _wid()
        ebase = wid * _EPT
        pltpu.sync_copy(dst_hbm.at[pl.ds(ebase, _EPT)], dst_v)
        for h in range(H):
            pltpu.sync_copy(sT_hbm.at[h, pl.ds(ebase, _EPT)], s_v)
            pltpu.sync_copy(den_hbm.at[h], den_v)

            @plsc.parallel_loop(0, _EPT // 16, unroll=8)
            def body(i):
                sl = pl.ds(i * 16, 16)
                d = plsc.load_gather(den_v, [dst_v[sl]])
                w_v[sl] = s_v[sl] / (d + jnp.float32(1e-16))
            pltpu.sync_copy(w_v, wT_hbm.at[h, pl.ds(ebase, _EPT)])

    return k


def _make_message_kernel(C, H128):
    @functools.partial(
        pl.kernel,
        out_type=jax.ShapeDtypeStruct((2, _NP, 128), jnp.float32),
        mesh=_mesh,
        compiler_params=pltpu.CompilerParams(use_tc_tiling_on_sc=False, needs_layout_passes=False),
        scratch_types=[
            pltpu.VMEM((_BPG, _EB), jnp.int32),
            pltpu.VMEM((_BPG, _EB), jnp.int32),
            pltpu.VMEM((H128, _EG), jnp.float32),
            pltpu.VMEM((_EB, 128), jnp.float32),
            pltpu.VMEM((_EB, 128), jnp.float32),
            pltpu.VMEM((_EB, 128), jnp.float32),
            pltpu.VMEM_SHARED((_NP, 128), jnp.float32),
            pltpu.SemaphoreType.DMA,
            pltpu.SemaphoreType.DMA,
        ],
    )
    def k(h_hbm, wT_hbm, src3_hbm, dst3_hbm, opart_hbm,
          src_g, dst_g, w_v, rows_a, rows_b, rows_o, acc_sh, sem_a, sem_b):
        cid = lax.axis_index("c")
        sid = lax.axis_index("s")
        wid = cid * 16 + sid
        ebase = wid * _EPT

        _zero2d(rows_a, _EB)
        r0 = sid * _LROWS

        def zbody(j, _):
            pltpu.sync_copy(rows_a, acc_sh.at[pl.ds(r0 + j * _EB, _EB)])
            return 0

        lax.fori_loop(0, _LROWS // _EB, zbody, 0)
        plsc.subcore_barrier()

        def scale_and_scatter(jb, rows_v):
            @plsc.parallel_loop(0, _EB, unroll=4)
            def edge(l):
                e = jb * _EB + l
                idxe = jnp.full((16,), 0, jnp.int32) + e
                for hh in range(H128):
                    ws = plsc.load_gather(
                        w_v, [jnp.full((16,), hh, jnp.int32), idxe])
                    for p in range(C // 16):
                        csl = pl.ds(hh * C + p * 16, 16)
                        rows_o[l, csl] = rows_v[l, csl] * ws

            pltpu.sync_copy(rows_o, acc_sh.at[dst_g.at[jb]], add=True)

        def group(g, _):
            pltpu.sync_copy(src3_hbm.at[wid, pl.ds(g * _BPG, _BPG)], src_g)
            pltpu.sync_copy(dst3_hbm.at[wid, pl.ds(g * _BPG, _BPG)], dst_g)
            for hh in range(H128):
                pltpu.sync_copy(wT_hbm.at[hh, pl.ds(ebase + g * _EG, _EG)],
                                w_v.at[hh])

            pltpu.async_copy(h_hbm.at[src_g.at[0]], rows_a, sem_a)

            def pair(p, _):
                j0 = 2 * p
                pltpu.async_copy(h_hbm.at[src_g.at[j0 + 1]], rows_b, sem_b)
                pltpu.make_async_copy(
                    h_hbm.at[src_g.at[j0]], rows_a, sem_a).wait()
                scale_and_scatter(j0, rows_a)
                pltpu.async_copy(h_hbm.at[src_g.at[j0 + 2]], rows_a, sem_a)
                pltpu.make_async_copy(
                    h_hbm.at[src_g.at[j0 + 1]], rows_b, sem_b).wait()
                scale_and_scatter(j0 + 1, rows_b)
                return 0

            lax.fori_loop(0, (_BPG - 1) // 2, pair, 0)
            pltpu.make_async_copy(
                h_hbm.at[src_g.at[_BPG - 1]], rows_a, sem_a).wait()
            scale_and_scatter(_BPG - 1, rows_a)
            return 0

        lax.fori_loop(0, _NG, group, 0)
        plsc.subcore_barrier()
        pltpu.sync_copy(acc_sh.at[pl.ds(r0, _LROWS)],
                        opart_hbm.at[cid, pl.ds(r0, _LROWS)])

    return k


def _zero2d(ref, nrows):
    def body(i, _):
        for p in range(8):
            ref[i, pl.ds(p * 16, 16)] = jnp.zeros((16,), jnp.float32)
        return 0
    lax.fori_loop(0, nrows, body, 0)


def _make_scalar_message_kernel():
    @functools.partial(
        pl.kernel,
        out_type=jax.ShapeDtypeStruct((_NW, _NP), jnp.float32),
        mesh=_mesh,
        compiler_params=pltpu.CompilerParams(use_tc_tiling_on_sc=False, needs_layout_passes=False),
        scratch_types=[
            pltpu.VMEM((_EPT,), jnp.int32),
            pltpu.VMEM((_EPT,), jnp.int32),
            pltpu.VMEM((_EPT,), jnp.float32),
            pltpu.VMEM((_NP,), jnp.float32),
            pltpu.VMEM((_NP,), jnp.float32),
        ],
    )
    def k(hcol_hbm, w_hbm, src_hbm, dst_hbm, opart_hbm,
          src_v, dst_v, w_v, hc_v, acc_v):
        wid = _wid()
        ebase = wid * _EPT
        pltpu.sync_copy(src_hbm.at[pl.ds(ebase, _EPT)], src_v)
        pltpu.sync_copy(dst_hbm.at[pl.ds(ebase, _EPT)], dst_v)
        pltpu.sync_copy(w_hbm.at[pl.ds(ebase, _EPT)], w_v)
        pltpu.sync_copy(hcol_hbm, hc_v)
        _zero_ref(acc_v, _NP)

        def body(i, _):
            sl = pl.ds(i * 16, 16)
            v = plsc.load_gather(hc_v, [src_v[sl]]) * w_v[sl]
            plsc.addupdate_scatter(acc_v, [dst_v[sl]], v)
            return 0

        lax.fori_loop(0, _EPT // 16, body, 0, unroll=8)
        pltpu.sync_copy(acc_v, opart_hbm.at[wid])

    return k


_logits8 = _make_logits_kernel(8)
_logits1 = _make_logits_kernel(1)
_weights8 = _make_weights_kernel(8)
_weights1 = _make_weights_kernel(1)
_msg_by_c = {128: _make_message_kernel(128, 1),
             64: _make_message_kernel(64, 2),
             32: _make_message_kernel(32, 4)}
_msg_scalar = _make_scalar_message_kernel()


def _expand_att(att):
    h, c = att.shape
    rows = np.arange(h * c)
    cols = np.repeat(np.arange(h), c)
    m = jnp.zeros((h * c, 128), jnp.float32)
    return m.at[rows, cols].set(att.reshape(-1))


def kernel(x, edge_index, edge_attr, params):
    src = edge_index[0].astype(jnp.int32)
    dst = edge_index[1].astype(jnp.int32)
    ea = edge_attr[:, 0].astype(jnp.float32)
    src3 = src.reshape(_NW, _NBLK, _EB)
    dst3 = dst.reshape(_NW, _NBLK, _EB)

    xin = jnp.pad(x, ((0, _NP - _N), (0, 0)))
    for i in range(4):
        H, C = _HEADS[i], _OUTC[i]
        hc = H * C
        w_mat = params['W%d' % i]
        if hc < 128:
            w_mat = jnp.pad(w_mat, ((0, 0), (0, 128 - hc)))
        asrc_m = _expand_att(params['att_src%d' % i])
        adst_m = _expand_att(params['att_dst%d' % i])
        if hc < 128:
            asrc_m = jnp.pad(asrc_m, ((0, 128 - hc), (0, 0)))
            adst_m = jnp.pad(adst_m, ((0, 128 - hc), (0, 0)))
        h_full, a_src, a_dst = _tc_matmul(xin, w_mat, asrc_m, adst_m)

        ae_m = _expand_att(params['att_e%d' % i])
        we_pad = jnp.pad(params['We%d' % i], ((0, 7), (0, 0)))
        if hc < 128:
            ae_m = jnp.pad(ae_m, ((0, 128 - hc), (0, 0)))
            we_pad = jnp.pad(we_pad, ((0, 0), (0, 128 - hc)))
        c16 = _tc_cvec(we_pad, ae_m)[0, :16]

        asrcT = a_src[:, :H].T + 0.0
        adstT = a_dst[:, :H].T + 0.0

        logits_k = _logits8 if H == 8 else _logits1
        weights_k = _weights8 if H == 8 else _weights1
        sT, dparts = logits_k(src, dst, asrcT, adstT, ea, c16)
        denom = _tc_sum_parts(dparts)
        wT = weights_k(dst, sT, denom)

        if i < 3:
            msg_k = _msg_by_c[C]
            h128 = 128 // C
            p0_chunks, p1_chunks = [], []
            for j in range(hc // 128):
                h_chunk = h_full[:, j * 128:(j + 1) * 128] + 0.0
                w_chunk = wT[j * h128:(j + 1) * h128] + 0.0
                opart = msg_k(h_chunk, w_chunk, src3, dst3)
                p0_chunks.append(opart[0])
                p1_chunks.append(opart[1])
            p0 = jnp.concatenate(p0_chunks, axis=1)
            p1 = jnp.concatenate(p1_chunks, axis=1)
            b2d = params['b%d' % i][None, :]
            xin = _tc_combine(p0, p1, b2d, True)
        else:
            hcol = h_full[:, 0] + 0.0
            parts = _msg_scalar(hcol, wT[0], src, dst)
            b2d = jnp.full((1, 128), params['b%d' % i][0], jnp.float32)
            out2d = _tc_sum32_bias(parts.reshape(_NW, _NP // 128, 128), b2d)
            return out2d.reshape(_NP)[:_N, None]
